# Initial kernel scaffold; baseline (speedup 1.0000x reference)
import functools

import jax
import jax.numpy as jnp
from jax import lax
from jax.experimental import pallas as pl
from jax.experimental.pallas import tpu as pltpu
from jax.experimental.pallas import tpu_sc as plsc

_F = 64
_D = 1024
_NC = 2
_NS = 16
_NW = _NC * _NS
_GW = 80
_RB = 1024


def _sc_gather(table, idx):
    n = idx.shape[0]
    per_w = n // _NW
    kc = per_w // _GW
    assert per_w % _GW == 0 and n % _NW == 0
    mesh = plsc.VectorSubcoreMesh(core_axis_name="c", subcore_axis_name="s")

    @functools.partial(
        pl.kernel,
        mesh=mesh,
        out_type=jax.ShapeDtypeStruct((n, _F), jnp.float32),
        scratch_types=[
            pltpu.VMEM((per_w,), jnp.int32),
            pltpu.VMEM((_GW, _F), jnp.float32),
            pltpu.SemaphoreType.DMA,
        ],
    )
    def k(tab_hbm, idx_hbm, out_hbm, idx_v, rows_v, sem):
        wid = lax.axis_index("s") * _NC + lax.axis_index("c")
        base = wid * per_w
        pltpu.sync_copy(idx_hbm.at[pl.ds(base, per_w)], idx_v)

        @pl.loop(0, kc)
        def _(j):
            off = j * _GW
            pltpu.async_copy(
                tab_hbm.at[idx_v.at[pl.ds(off, _GW)]], rows_v, sem
            ).wait()
            pltpu.sync_copy(rows_v, out_hbm.at[pl.ds(base + off, _GW)])

    return k(table, idx)


def _tc_project(emb, w, b2d):
    n = emb.shape[0]
    nb = n // _RB

    def body(e_ref, w_ref, b_ref, o_ref):
        o_ref[...] = lax.dot_general(
            e_ref[...], w_ref[...], (((1,), (1,)), ((), ())),
            preferred_element_type=jnp.float32,
        ) + b_ref[...]

    return pl.pallas_call(
        body,
        grid=(nb,),
        in_specs=[
            pl.BlockSpec((_RB, _F), lambda i: (i, 0)),
            pl.BlockSpec((_D, _F), lambda i: (0, 0)),
            pl.BlockSpec((1, _D), lambda i: (0, 0)),
        ],
        out_specs=pl.BlockSpec((_RB, _D), lambda i: (i, 0)),
        out_shape=jax.ShapeDtypeStruct((n, _D), jnp.float32),
    )(emb, w, b2d)


def kernel(x, main_embed, W_proj, b_proj):
    bsz, seq = x.shape
    n = bsz * seq
    idx = x.reshape(n).astype(jnp.int32)
    emb = _sc_gather(main_embed, idx)
    out = _tc_project(emb, W_proj, b_proj.reshape(1, _D))
    return out.reshape(bsz, seq, _D)


# speedup vs baseline: 1.0867x; 1.0867x over previous
import functools

import jax
import jax.numpy as jnp
from jax import lax
from jax.experimental import pallas as pl
from jax.experimental.pallas import tpu as pltpu
from jax.experimental.pallas import tpu_sc as plsc

_F = 64
_D = 1024
_NC = 2
_NS = 16
_NW = _NC * _NS
_GW = 80
_RB = 1024


def _sc_gather(table, idx):
    n = idx.shape[0]
    per_w = n // _NW
    kc = per_w // _GW
    assert per_w % _GW == 0 and n % _NW == 0
    mesh = plsc.VectorSubcoreMesh(core_axis_name="c", subcore_axis_name="s")

    @functools.partial(
        pl.kernel,
        mesh=mesh,
        compiler_params=pltpu.CompilerParams(use_tc_tiling_on_sc=False),
        out_type=jax.ShapeDtypeStruct((n, _F), jnp.float32),
        scratch_types=[
            pltpu.VMEM((per_w,), jnp.int32),
            pltpu.VMEM((_GW, _F), jnp.float32),
            pltpu.SemaphoreType.DMA,
        ],
    )
    def k(tab_hbm, idx_hbm, out_hbm, idx_v, rows_v, sem):
        wid = lax.axis_index("s") * _NC + lax.axis_index("c")
        base = wid * per_w
        pltpu.sync_copy(idx_hbm.at[pl.ds(base, per_w)], idx_v)

        @pl.loop(0, kc)
        def _(j):
            off = j * _GW
            pltpu.async_copy(
                tab_hbm.at[idx_v.at[pl.ds(off, _GW)]], rows_v, sem
            ).wait()
            pltpu.sync_copy(rows_v, out_hbm.at[pl.ds(base + off, _GW)])

    return k(table, idx)


def _tc_project(emb, w, b2d):
    n = emb.shape[0]
    nb = n // _RB

    def body(e_ref, w_ref, b_ref, o_ref):
        o_ref[...] = lax.dot_general(
            e_ref[...], w_ref[...], (((1,), (1,)), ((), ())),
            preferred_element_type=jnp.float32,
        ) + b_ref[...]

    return pl.pallas_call(
        body,
        grid=(nb,),
        in_specs=[
            pl.BlockSpec((_RB, _F), lambda i: (i, 0)),
            pl.BlockSpec((_D, _F), lambda i: (0, 0)),
            pl.BlockSpec((1, _D), lambda i: (0, 0)),
        ],
        out_specs=pl.BlockSpec((_RB, _D), lambda i: (i, 0)),
        out_shape=jax.ShapeDtypeStruct((n, _D), jnp.float32),
    )(emb, w, b2d)


def kernel(x, main_embed, W_proj, b_proj):
    bsz, seq = x.shape
    n = bsz * seq
    idx = x.reshape(n).astype(jnp.int32)
    emb = _sc_gather(main_embed, idx)
    out = _tc_project(emb, W_proj, b_proj.reshape(1, _D))
    return out.reshape(bsz, seq, _D)
